# baseline (speedup 1.0000x reference)
import functools

import jax
import jax.numpy as jnp
import numpy as np
from jax import lax
from jax.experimental import pallas as pl
from jax.experimental.pallas import tpu as pltpu
from jax.experimental.pallas import tpu_sc as plsc

NC = 2
NS = 16
NW = NC * NS

DIM = 64

_H_COEFFS = (
    0.6931472757981448,
    0.12499834228441935,
    -0.0052036006182432275,
    0.0003420800293110525,
    -2.3602684200345206e-05,
    1.3255080148215367e-06,
    -3.98244079740464e-08,
)


def _sc_gather(titems, cidx, nidx, big):
    B = titems.shape[0]
    F = cidx.shape[0]
    t_per_w = B // NW
    f_per_w = F // NW
    CHUNK = min(320, f_per_w)
    n_chunks = f_per_w // CHUNK

    mesh = plsc.VectorSubcoreMesh(core_axis_name="c", subcore_axis_name="s")

    @functools.partial(
        pl.kernel,
        mesh=mesh,
        compiler_params=pltpu.CompilerParams(needs_layout_passes=False),
        out_type=(
            jax.ShapeDtypeStruct((B, 2 * DIM), jnp.float32),
            jax.ShapeDtypeStruct((F, 2 * DIM), jnp.float32),
            jax.ShapeDtypeStruct((F, 2 * DIM), jnp.float32),
        ),
        scratch_types=[
            pltpu.VMEM((CHUNK,), jnp.int32),
            pltpu.VMEM((CHUNK,), jnp.int32),
            pltpu.VMEM((CHUNK, 2 * DIM), jnp.float32),
            pltpu.VMEM((CHUNK, 2 * DIM), jnp.float32),
            pltpu.SemaphoreType.DMA,
            pltpu.SemaphoreType.DMA,
        ],
    )
    def gather_kernel(ti_h, ci_h, ni_h, big_h,
                      tout_h, cout_h, nout_h,
                      idx0, idx1, rows0, rows1, sem0, sem1):
        wid = lax.axis_index("s") * NC + lax.axis_index("c")
        tb = wid * t_per_w
        fb = wid * f_per_w

        tasks = [(ti_h, tout_h, tb, t_per_w)]
        tasks += [(ci_h, cout_h, fb + i * CHUNK, CHUNK)
                  for i in range(n_chunks)]
        tasks += [(ni_h, nout_h, fb + i * CHUNK, CHUNK)
                  for i in range(n_chunks)]

        idxb = (idx0, idx1)
        rowsb = (rows0, rows1)
        semb = (sem0, sem1)

        def start(k):
            src_h, _, off, cnt = tasks[k]
            b = k % 2
            pltpu.sync_copy(src_h.at[pl.ds(off, cnt)],
                            idxb[b].at[pl.ds(0, cnt)])
            return pltpu.async_copy(
                big_h.at[idxb[b].at[pl.ds(0, cnt)]],
                rowsb[b].at[pl.ds(0, cnt)], semb[b])

        pending = start(0)
        for k in range(len(tasks)):
            nxt = start(k + 1) if k + 1 < len(tasks) else None
            pending.wait()
            _, out_h, off, cnt = tasks[k]
            pltpu.sync_copy(rowsb[k % 2].at[pl.ds(0, cnt)],
                            out_h.at[pl.ds(off, cnt)])
            pending = nxt

    return gather_kernel(titems, cidx, nidx, big)


def _tc_loss(tp, cp, nv_p, ctx, negs):
    B = tp.shape[0]
    NB = 32
    steps = B // NB
    R = NB * ctx
    C = NB + NB * negs

    rows_b = np.arange(R)[:, None] // ctx
    cols = np.arange(C)[None, :]
    is_t = cols < NB
    cols_b = np.where(is_t, cols, (cols - NB) // negs)
    mask = (rows_b == cols_b).astype(np.float32)
    msign = np.where(is_t, -1.0, 1.0).astype(np.float32)
    a2 = jnp.asarray(mask * msign * 0.5)

    def body(a2_ref, tp_ref, cp_ref, np_ref, out_ref):
        cv = cp_ref[:, DIM:]
        tv = tp_ref[:, :DIM]
        nv = np_ref[:, :DIM]
        allt = jnp.concatenate([tv, nv], axis=0)
        g = lax.dot_general(cv, allt, (((1,), (1,)), ((), ())),
                            preferred_element_type=jnp.float32)
        a2v = a2_ref[...]
        t = g * g
        h = jnp.float32(_H_COEFFS[6])
        for c in _H_COEFFS[5::-1]:
            h = h * t + jnp.float32(c)
        contrib = g * a2v + (2.0 * jnp.abs(a2v)) * h

        @pl.when(pl.program_id(0) == 0)
        def _():
            out_ref[...] = jnp.zeros((R, C), jnp.float32)

        out_ref[...] += contrib

    out = pl.pallas_call(
        body,
        grid=(steps,),
        in_specs=[
            pl.BlockSpec((R, C), lambda i: (0, 0)),
            pl.BlockSpec((NB, 2 * DIM), lambda i: (i, 0)),
            pl.BlockSpec((R, 2 * DIM), lambda i: (i, 0)),
            pl.BlockSpec((R, 2 * DIM), lambda i: (i, 0)),
        ],
        out_specs=pl.BlockSpec((R, C), lambda i: (0, 0)),
        out_shape=jax.ShapeDtypeStruct((R, C), jnp.float32),
    )(a2, tp, cp, nv_p)
    return jnp.sum(out)


def kernel(titems, citems, nitems, tvectors, cvectors):
    B, ctx = citems.shape
    negs = nitems.shape[1]
    big = jnp.concatenate([tvectors, cvectors], axis=1)
    cidx = citems.reshape(-1)
    nidx = nitems.reshape(-1)
    PHASES = 4
    H = B // PHASES
    FH = H * ctx
    total = jnp.float32(0)
    for ph in range(PHASES):
        tp, cp, nv_p = _sc_gather(
            titems[ph * H:(ph + 1) * H],
            cidx[ph * FH:(ph + 1) * FH],
            nidx[ph * FH:(ph + 1) * FH], big)
        total = total + _tc_loss(tp, cp, nv_p, ctx, negs)
    return total / B


# speedup vs baseline: 1.0034x; 1.0034x over previous
import functools

import jax
import jax.numpy as jnp
import numpy as np
from jax import lax
from jax.experimental import pallas as pl
from jax.experimental.pallas import tpu as pltpu
from jax.experimental.pallas import tpu_sc as plsc

NC = 2
NS = 16
NW = NC * NS

DIM = 64

_H_COEFFS = (
    0.6931472757981448,
    0.12499834228441935,
    -0.0052036006182432275,
    0.0003420800293110525,
    -2.3602684200345206e-05,
    1.3255080148215367e-06,
    -3.98244079740464e-08,
)


def _sc_gather(titems, cidx, nidx, big):
    B = titems.shape[0]
    F = cidx.shape[0]
    t_per_w = B // NW
    f_per_w = F // NW
    CHUNK = 256
    n_chunks = f_per_w // CHUNK

    mesh = plsc.VectorSubcoreMesh(core_axis_name="c", subcore_axis_name="s")

    @functools.partial(
        pl.kernel,
        mesh=mesh,
        compiler_params=pltpu.CompilerParams(needs_layout_passes=False),
        out_type=(
            jax.ShapeDtypeStruct((B, 2 * DIM), jnp.float32),
            jax.ShapeDtypeStruct((F, 2 * DIM), jnp.float32),
            jax.ShapeDtypeStruct((F, 2 * DIM), jnp.float32),
        ),
        scratch_types=[
            pltpu.VMEM((CHUNK,), jnp.int32),
            pltpu.VMEM((CHUNK,), jnp.int32),
            pltpu.VMEM((CHUNK, 2 * DIM), jnp.float32),
            pltpu.VMEM((CHUNK, 2 * DIM), jnp.float32),
            pltpu.SemaphoreType.DMA,
            pltpu.SemaphoreType.DMA,
        ],
    )
    def gather_kernel(ti_h, ci_h, ni_h, big_h,
                      tout_h, cout_h, nout_h,
                      idx0, idx1, rows0, rows1, sem0, sem1):
        wid = lax.axis_index("s") * NC + lax.axis_index("c")
        tb = wid * t_per_w
        fb = wid * f_per_w

        tasks = [(ti_h, tout_h, tb, t_per_w)]
        tasks += [(ci_h, cout_h, fb + i * CHUNK, CHUNK)
                  for i in range(n_chunks)]
        tasks += [(ni_h, nout_h, fb + i * CHUNK, CHUNK)
                  for i in range(n_chunks)]

        idxb = (idx0, idx1)
        rowsb = (rows0, rows1)
        semb = (sem0, sem1)

        def start(k):
            src_h, _, off, cnt = tasks[k]
            b = k % 2
            pltpu.sync_copy(src_h.at[pl.ds(off, cnt)],
                            idxb[b].at[pl.ds(0, cnt)])
            return pltpu.async_copy(
                big_h.at[idxb[b].at[pl.ds(0, cnt)]],
                rowsb[b].at[pl.ds(0, cnt)], semb[b])

        pending = start(0)
        for k in range(len(tasks)):
            nxt = start(k + 1) if k + 1 < len(tasks) else None
            pending.wait()
            _, out_h, off, cnt = tasks[k]
            pltpu.sync_copy(rowsb[k % 2].at[pl.ds(0, cnt)],
                            out_h.at[pl.ds(off, cnt)])
            pending = nxt

    return gather_kernel(titems, cidx, nidx, big)


def _tc_loss(tp, cp, nv_p, ctx, negs):
    B = tp.shape[0]
    NB = 32
    steps = B // NB
    R = NB * ctx
    C = NB + NB * negs

    rows_b = np.arange(R)[:, None] // ctx
    cols = np.arange(C)[None, :]
    is_t = cols < NB
    cols_b = np.where(is_t, cols, (cols - NB) // negs)
    mask = (rows_b == cols_b).astype(np.float32)
    msign = np.where(is_t, -1.0, 1.0).astype(np.float32)
    a2 = jnp.asarray(mask * msign * 0.5)

    def body(a2_ref, tp_ref, cp_ref, np_ref, out_ref):
        cv = cp_ref[:, DIM:]
        tv = tp_ref[:, :DIM]
        nv = np_ref[:, :DIM]
        allt = jnp.concatenate([tv, nv], axis=0)
        g = lax.dot_general(cv, allt, (((1,), (1,)), ((), ())),
                            preferred_element_type=jnp.float32)
        a2v = a2_ref[...]
        t = g * g
        h = jnp.float32(_H_COEFFS[6])
        for c in _H_COEFFS[5::-1]:
            h = h * t + jnp.float32(c)
        contrib = g * a2v + (2.0 * jnp.abs(a2v)) * h

        @pl.when(pl.program_id(0) == 0)
        def _():
            out_ref[...] = jnp.zeros((R, C), jnp.float32)

        out_ref[...] += contrib

    out = pl.pallas_call(
        body,
        grid=(steps,),
        in_specs=[
            pl.BlockSpec((R, C), lambda i: (0, 0)),
            pl.BlockSpec((NB, 2 * DIM), lambda i: (i, 0)),
            pl.BlockSpec((R, 2 * DIM), lambda i: (i, 0)),
            pl.BlockSpec((R, 2 * DIM), lambda i: (i, 0)),
        ],
        out_specs=pl.BlockSpec((R, C), lambda i: (0, 0)),
        out_shape=jax.ShapeDtypeStruct((R, C), jnp.float32),
    )(a2, tp, cp, nv_p)
    return jnp.sum(out)


def kernel(titems, citems, nitems, tvectors, cvectors):
    B, ctx = citems.shape
    negs = nitems.shape[1]
    big = jnp.concatenate([tvectors, cvectors], axis=1)
    cidx = citems.reshape(-1)
    nidx = nitems.reshape(-1)
    PHASES = 2
    H = B // PHASES
    FH = H * ctx
    total = jnp.float32(0)
    for ph in range(PHASES):
        tp, cp, nv_p = _sc_gather(
            titems[ph * H:(ph + 1) * H],
            cidx[ph * FH:(ph + 1) * FH],
            nidx[ph * FH:(ph + 1) * FH], big)
        total = total + _tc_loss(tp, cp, nv_p, ctx, negs)
    return total / B
